# baseline (speedup 1.0000x reference)
import numpy as np
import jax
import jax.numpy as jnp
from jax import lax
from jax.experimental import pallas as pl
from jax.experimental.pallas import tpu as pltpu
from jax.experimental.pallas import tpu_sc as plsc

_NUM_LEVEL = 16
_LEVEL_DIM = 2
_BASE_RES = 16
_MAX_PARAMS = 2 ** 19
_N = 65536


def _grid_offsets():
    offs = [0]
    off = 0
    for i in range(_NUM_LEVEL):
        res = int(np.ceil(_BASE_RES * 2.0 ** i))
        p = min(_MAX_PARAMS, res ** 3)
        p = int(np.ceil(p / 8) * 8)
        off += p
        offs.append(off)
    return offs


_OFFSETS = _grid_offsets()
_TOTAL_PARAMS = _OFFSETS[-1]
_P1 = np.uint32(2654435761).astype(np.int32)
_P2 = np.uint32(805459861).astype(np.int32)

_NC, _NS = 2, 16
_NW = _NC * _NS
_PPW = _N // _NW
_CH = 512
_NCH = _PPW // _CH


def _body(xs_h, ys_h, zs_h, emb_h, out_h,
          xv, yv, zv, fxb, fyb, fzb,
          g0, g1, g2, g3, g4, g5, g6, g7,
          l0, l1, l2, l3, l4, l5, l6, l7,
          r0, r1, r2, r3, r4, r5, r6, r7,
          outv, sem):
    grp_bufs = (g0, g1, g2, g3, g4, g5, g6, g7)
    low_bufs = (l0, l1, l2, l3, l4, l5, l6, l7)
    row_bufs = (r0, r1, r2, r3, r4, r5, r6, r7)
    cid = lax.axis_index("c")
    sid = lax.axis_index("s")
    wid = sid * _NC + cid

    iot = lax.iota(jnp.int32, 16)
    half = lax.shift_right_logical(iot, 1)
    par = lax.bitwise_and(iot, 1)

    def process_level(lvl, scale_f, off, mask, use_hash, res):
        def p1(i, c):
            s = i * 16
            x = xv[pl.ds(s, 16)]
            y = yv[pl.ds(s, 16)]
            z = zv[pl.ds(s, 16)]
            px = ((x + 1.0) * 0.5) * scale_f
            py = ((y + 1.0) * 0.5) * scale_f
            pz = ((z + 1.0) * 0.5) * scale_f
            bx = px.astype(jnp.int32)
            by = py.astype(jnp.int32)
            bz = pz.astype(jnp.int32)
            fxb[pl.ds(s, 16)] = px - bx.astype(jnp.float32)
            fyb[pl.ds(s, 16)] = py - by.astype(jnp.float32)
            fzb[pl.ds(s, 16)] = pz - bz.astype(jnp.float32)
            if use_hash:
                h0a = bx
                h0b = bx + 1
                h1a = by * _P1
                h1b = h1a + _P1
                h2a = bz * _P2
                h2b = h2a + _P2
            else:
                h0a = bx
                h0b = bx + 1
                h1a = by * res
                h1b = h1a + res
                h2a = bz * (res * res)
                h2b = h2a + (res * res)
            for c8 in range(8):
                e0 = h0b if (c8 & 1) else h0a
                e1 = h1b if (c8 & 2) else h1a
                e2 = h2b if (c8 & 4) else h2a
                if use_hash:
                    idx = lax.bitwise_xor(lax.bitwise_xor(e0, e1), e2)
                else:
                    idx = e0 + e1 + e2
                idx = lax.bitwise_and(idx, mask) + off
                grp_bufs[c8][pl.ds(s, 16)] = lax.shift_right_logical(idx, 2)
                low_bufs[c8][pl.ds(s, 16)] = lax.bitwise_and(idx, 3)
            return c

        lax.fori_loop(0, _CH // 16, p1, 0, unroll=False)

        descs = [pltpu.async_copy(emb_h.at[grp_bufs[c8]], row_bufs[c8], sem)
                 for c8 in range(8)]
        for d in descs:
            d.wait()

        colv = par + (2 * lvl)

        def p2(g, c):
            ridx = half + g * 8
            fxi = plsc.load_gather(fxb, [ridx])
            fyi = plsc.load_gather(fyb, [ridx])
            fzi = plsc.load_gather(fzb, [ridx])
            gxi = 1.0 - fxi
            gyi = 1.0 - fyi
            gzi = 1.0 - fzi
            w00 = gxi * gyi
            w10 = fxi * gyi
            w01 = gxi * fyi
            w11 = fxi * fyi
            ws = (w00 * gzi, w10 * gzi, w01 * gzi, w11 * gzi,
                  w00 * fzi, w10 * fzi, w01 * fzi, w11 * fzi)
            acc = None
            for c8 in range(8):
                li = plsc.load_gather(low_bufs[c8], [ridx])
                col = lax.shift_left(li, 1) + par
                v = plsc.load_gather(row_bufs[c8], [ridx, col])
                acc = ws[c8] * v if acc is None else acc + ws[c8] * v
            plsc.store_scatter(outv, [ridx, colv], acc)
            return c

        lax.fori_loop(0, _CH // 8, p2, 0, unroll=False)

    for chunk in range(_NCH):
        base = wid * _PPW + chunk * _CH
        pltpu.sync_copy(xs_h.at[pl.ds(base, _CH)], xv)
        pltpu.sync_copy(ys_h.at[pl.ds(base, _CH)], yv)
        pltpu.sync_copy(zs_h.at[pl.ds(base, _CH)], zv)

        for lvl in range(3):
            res = _BASE_RES * (2 ** lvl)
            scale_f = jnp.float32(res - 1)
            size = _OFFSETS[lvl + 1] - _OFFSETS[lvl]
            process_level(lvl, scale_f, jnp.int32(_OFFSETS[lvl]),
                          jnp.int32(size - 1), False, jnp.int32(res))

        def lev_body(lvl, c):
            two = lax.shift_left(jnp.int32(1), lvl)
            scale_f = (two * _BASE_RES - 1).astype(jnp.float32)
            off = jnp.int32(_OFFSETS[3]) + (lvl - 3) * jnp.int32(_MAX_PARAMS)
            process_level(lvl, scale_f, off, jnp.int32(_MAX_PARAMS - 1),
                          True, None)
            return c

        lax.fori_loop(3, _NUM_LEVEL, lev_body, 0, unroll=False)

        pltpu.sync_copy(outv, out_h.at[pl.ds(base, _CH)])


_mesh = plsc.VectorSubcoreMesh(core_axis_name="c", subcore_axis_name="s",
                               num_cores=_NC, num_subcores=_NS)

_scratch = (
    [pltpu.VMEM((_CH,), jnp.float32) for _ in range(6)]
    + [pltpu.VMEM((_CH,), jnp.int32) for _ in range(8)]
    + [pltpu.VMEM((_CH,), jnp.int32) for _ in range(8)]
    + [pltpu.VMEM((_CH, 8), jnp.float32) for _ in range(8)]
    + [pltpu.VMEM((_CH, 2 * _NUM_LEVEL), jnp.float32)]
    + [pltpu.SemaphoreType.DMA]
)

_grid_kernel = pl.kernel(
    _body,
    out_type=jax.ShapeDtypeStruct((_N, 2 * _NUM_LEVEL), jnp.float32),
    mesh=_mesh,
    scratch_types=_scratch,
    compiler_params=pltpu.CompilerParams(needs_layout_passes=False,
                                         use_tc_tiling_on_sc=False),
)


def kernel(input_means, embeddings):
    ori_shape = input_means.shape[:-1]
    pts = input_means.reshape(-1, 3).T
    e64 = lax.bitcast_convert_type(embeddings, jnp.uint64)
    emb8 = lax.bitcast_convert_type(e64.reshape(-1, 4), jnp.float32)
    emb8 = emb8.reshape(-1, 8)
    out = _grid_kernel(pts[0], pts[1], pts[2], emb8)
    return out.reshape(ori_shape + (2 * _NUM_LEVEL,))


# speedup vs baseline: 1.0004x; 1.0004x over previous
import numpy as np
import jax
import jax.numpy as jnp
from jax import lax
from jax.experimental import pallas as pl
from jax.experimental.pallas import tpu as pltpu
from jax.experimental.pallas import tpu_sc as plsc

_NUM_LEVEL = 16
_LEVEL_DIM = 2
_BASE_RES = 16
_MAX_PARAMS = 2 ** 19
_N = 65536


def _grid_offsets():
    offs = [0]
    off = 0
    for i in range(_NUM_LEVEL):
        res = int(np.ceil(_BASE_RES * 2.0 ** i))
        p = min(_MAX_PARAMS, res ** 3)
        p = int(np.ceil(p / 8) * 8)
        off += p
        offs.append(off)
    return offs


_OFFSETS = _grid_offsets()
_TOTAL_PARAMS = _OFFSETS[-1]
_P1 = np.uint32(2654435761).astype(np.int32)
_P2 = np.uint32(805459861).astype(np.int32)

_NC, _NS = 2, 16
_NW = _NC * _NS
_PPW = _N // _NW
_CH = 512
_NCH = _PPW // _CH


def _body(xs_h, ys_h, zs_h, emb_h, out_h,
          xv, yv, zv, fxb, fyb, fzb,
          g0, g1, g2, g3, g4, g5, g6, g7,
          l0, l1, l2, l3, l4, l5, l6, l7,
          r0, r1, r2, r3, r4, r5, r6, r7,
          outv, sem):
    grp_bufs = (g0, g1, g2, g3, g4, g5, g6, g7)
    low_bufs = (l0, l1, l2, l3, l4, l5, l6, l7)
    row_bufs = (r0, r1, r2, r3, r4, r5, r6, r7)
    cid = lax.axis_index("c")
    sid = lax.axis_index("s")
    wid = sid * _NC + cid

    iot = lax.iota(jnp.int32, 16)
    half = lax.shift_right_logical(iot, 1)
    par = lax.bitwise_and(iot, 1)

    def process_level(lvl, scale_f, off, mask, use_hash, res):
        def p1(i, c):
            s = i * 16
            x = xv[pl.ds(s, 16)]
            y = yv[pl.ds(s, 16)]
            z = zv[pl.ds(s, 16)]
            px = ((x + 1.0) * 0.5) * scale_f
            py = ((y + 1.0) * 0.5) * scale_f
            pz = ((z + 1.0) * 0.5) * scale_f
            bx = px.astype(jnp.int32)
            by = py.astype(jnp.int32)
            bz = pz.astype(jnp.int32)
            fxb[pl.ds(s, 16)] = px - bx.astype(jnp.float32)
            fyb[pl.ds(s, 16)] = py - by.astype(jnp.float32)
            fzb[pl.ds(s, 16)] = pz - bz.astype(jnp.float32)
            if use_hash:
                h0a = bx
                h0b = bx + 1
                h1a = by * _P1
                h1b = h1a + _P1
                h2a = bz * _P2
                h2b = h2a + _P2
            else:
                h0a = bx
                h0b = bx + 1
                h1a = by * res
                h1b = h1a + res
                h2a = bz * (res * res)
                h2b = h2a + (res * res)
            for c8 in range(8):
                e0 = h0b if (c8 & 1) else h0a
                e1 = h1b if (c8 & 2) else h1a
                e2 = h2b if (c8 & 4) else h2a
                if use_hash:
                    idx = lax.bitwise_xor(lax.bitwise_xor(e0, e1), e2)
                else:
                    idx = e0 + e1 + e2
                idx = lax.bitwise_and(idx, mask) + off
                grp_bufs[c8][pl.ds(s, 16)] = lax.shift_right_logical(idx, 2)
                low_bufs[c8][pl.ds(s, 16)] = lax.bitwise_and(idx, 3)
            return c

        lax.fori_loop(0, _CH // 16, p1, 0, unroll=False)

        descs = [pltpu.async_copy(emb_h.at[grp_bufs[c8]], row_bufs[c8], sem)
                 for c8 in range(8)]
        for d in descs:
            d.wait()

        colv = par + (2 * lvl)

        def p2(g, c):
            ridx = half + g * 8
            fxi = plsc.load_gather(fxb, [ridx])
            fyi = plsc.load_gather(fyb, [ridx])
            fzi = plsc.load_gather(fzb, [ridx])
            gxi = 1.0 - fxi
            gyi = 1.0 - fyi
            gzi = 1.0 - fzi
            w00 = gxi * gyi
            w10 = fxi * gyi
            w01 = gxi * fyi
            w11 = fxi * fyi
            ws = (w00 * gzi, w10 * gzi, w01 * gzi, w11 * gzi,
                  w00 * fzi, w10 * fzi, w01 * fzi, w11 * fzi)
            acc = None
            for c8 in range(8):
                li = plsc.load_gather(low_bufs[c8], [ridx])
                col = lax.shift_left(li, 1) + par
                v = plsc.load_gather(row_bufs[c8], [ridx, col])
                acc = ws[c8] * v if acc is None else acc + ws[c8] * v
            plsc.store_scatter(outv, [ridx, colv], acc)
            return c

        lax.fori_loop(0, _CH // 8, p2, 0, unroll=False)

    for chunk in range(_NCH):
        base = wid * _PPW + chunk * _CH
        pltpu.sync_copy(xs_h.at[pl.ds(base, _CH)], xv)
        pltpu.sync_copy(ys_h.at[pl.ds(base, _CH)], yv)
        pltpu.sync_copy(zs_h.at[pl.ds(base, _CH)], zv)

        for lvl in range(3):
            res = _BASE_RES * (2 ** lvl)
            scale_f = jnp.float32(res - 1)
            size = _OFFSETS[lvl + 1] - _OFFSETS[lvl]
            process_level(lvl, scale_f, jnp.int32(_OFFSETS[lvl]),
                          jnp.int32(size - 1), False, jnp.int32(res))

        def lev_body(lvl, c):
            two = lax.shift_left(jnp.int32(1), lvl)
            scale_f = (two * _BASE_RES - 1).astype(jnp.float32)
            off = jnp.int32(_OFFSETS[3]) + (lvl - 3) * jnp.int32(_MAX_PARAMS)
            process_level(lvl, scale_f, off, jnp.int32(_MAX_PARAMS - 1),
                          True, None)
            return c

        lax.fori_loop(3, _NUM_LEVEL, lev_body, 0, unroll=False)

        pltpu.sync_copy(outv, out_h.at[pl.ds(base, _CH)])


_mesh = plsc.VectorSubcoreMesh(core_axis_name="c", subcore_axis_name="s",
                               num_cores=_NC, num_subcores=_NS)

_scratch = (
    [pltpu.VMEM((_CH,), jnp.float32) for _ in range(6)]
    + [pltpu.VMEM((_CH,), jnp.int32) for _ in range(8)]
    + [pltpu.VMEM((_CH,), jnp.int32) for _ in range(8)]
    + [pltpu.VMEM((_CH, 8), jnp.float32) for _ in range(8)]
    + [pltpu.VMEM((_CH, 2 * _NUM_LEVEL), jnp.float32)]
    + [pltpu.SemaphoreType.DMA]
)

_grid_kernel = pl.kernel(
    _body,
    out_type=jax.ShapeDtypeStruct((_N, 2 * _NUM_LEVEL), jnp.float32),
    mesh=_mesh,
    scratch_types=_scratch,
    compiler_params=pltpu.CompilerParams(needs_layout_passes=False,
                                         use_tc_tiling_on_sc=False),
)


def kernel(input_means, embeddings):
    ori_shape = input_means.shape[:-1]
    pts = input_means.reshape(-1, 3).T
    c0 = embeddings[:, 0]
    c1 = embeddings[:, 1]
    emb8 = jnp.stack([c0, c1], axis=-1).reshape(-1, 8)
    out = _grid_kernel(pts[0], pts[1], pts[2], emb8)
    return out.reshape(ori_shape + (2 * _NUM_LEVEL,))


# speedup vs baseline: 13.1838x; 13.1779x over previous
import numpy as np
import jax
import jax.numpy as jnp
from jax import lax
from jax.experimental import pallas as pl
from jax.experimental.pallas import tpu as pltpu
from jax.experimental.pallas import tpu_sc as plsc

_NUM_LEVEL = 16
_LEVEL_DIM = 2
_BASE_RES = 16
_MAX_PARAMS = 2 ** 19
_N = 65536


def _grid_offsets():
    offs = [0]
    off = 0
    for i in range(_NUM_LEVEL):
        res = int(np.ceil(_BASE_RES * 2.0 ** i))
        p = min(_MAX_PARAMS, res ** 3)
        p = int(np.ceil(p / 8) * 8)
        off += p
        offs.append(off)
    return offs


_OFFSETS = _grid_offsets()
_TOTAL_PARAMS = _OFFSETS[-1]
_P1 = np.uint32(2654435761).astype(np.int32)
_P2 = np.uint32(805459861).astype(np.int32)

_NC, _NS = 2, 16
_NW = _NC * _NS
_PPW = _N // _NW
_CH = 512
_NCH = _PPW // _CH


def _body(xs_h, ys_h, zs_h, emb_h, out_h,
          xv, yv, zv, fxb, fyb, fzb,
          g0, g1, g2, g3, g4, g5, g6, g7,
          l0, l1, l2, l3, l4, l5, l6, l7,
          r0, r1, r2, r3, r4, r5, r6, r7,
          outv, sem):
    grp_bufs = (g0, g1, g2, g3, g4, g5, g6, g7)
    low_bufs = (l0, l1, l2, l3, l4, l5, l6, l7)
    row_bufs = (r0, r1, r2, r3, r4, r5, r6, r7)
    cid = lax.axis_index("c")
    sid = lax.axis_index("s")
    wid = sid * _NC + cid

    iot = lax.iota(jnp.int32, 16)
    half = lax.shift_right_logical(iot, 1)
    par = lax.bitwise_and(iot, 1)

    def process_level(lvl, scale_f, off, mask, use_hash, res):
        def p1(i, c):
            s = i * 16
            x = xv[pl.ds(s, 16)]
            y = yv[pl.ds(s, 16)]
            z = zv[pl.ds(s, 16)]
            px = ((x + 1.0) * 0.5) * scale_f
            py = ((y + 1.0) * 0.5) * scale_f
            pz = ((z + 1.0) * 0.5) * scale_f
            bx = px.astype(jnp.int32)
            by = py.astype(jnp.int32)
            bz = pz.astype(jnp.int32)
            fxb[pl.ds(s, 16)] = px - bx.astype(jnp.float32)
            fyb[pl.ds(s, 16)] = py - by.astype(jnp.float32)
            fzb[pl.ds(s, 16)] = pz - bz.astype(jnp.float32)
            if use_hash:
                h0a = bx
                h0b = bx + 1
                h1a = by * _P1
                h1b = h1a + _P1
                h2a = bz * _P2
                h2b = h2a + _P2
            else:
                h0a = bx
                h0b = bx + 1
                h1a = by * res
                h1b = h1a + res
                h2a = bz * (res * res)
                h2b = h2a + (res * res)
            for c8 in range(8):
                e0 = h0b if (c8 & 1) else h0a
                e1 = h1b if (c8 & 2) else h1a
                e2 = h2b if (c8 & 4) else h2a
                if use_hash:
                    idx = lax.bitwise_xor(lax.bitwise_xor(e0, e1), e2)
                else:
                    idx = e0 + e1 + e2
                idx = lax.bitwise_and(idx, mask) + off
                grp_bufs[c8][pl.ds(s, 16)] = lax.shift_right_logical(idx, 2)
                low_bufs[c8][pl.ds(s, 16)] = lax.bitwise_and(idx, 3)
            return c

        lax.fori_loop(0, _CH // 16, p1, 0, unroll=False)

        descs = [pltpu.async_copy(emb_h.at[grp_bufs[c8]], row_bufs[c8], sem)
                 for c8 in range(8)]
        for d in descs:
            d.wait()

        colv = par + (2 * lvl)

        def p2(g, c):
            ridx = half + g * 8
            fxi = plsc.load_gather(fxb, [ridx])
            fyi = plsc.load_gather(fyb, [ridx])
            fzi = plsc.load_gather(fzb, [ridx])
            gxi = 1.0 - fxi
            gyi = 1.0 - fyi
            gzi = 1.0 - fzi
            w00 = gxi * gyi
            w10 = fxi * gyi
            w01 = gxi * fyi
            w11 = fxi * fyi
            ws = (w00 * gzi, w10 * gzi, w01 * gzi, w11 * gzi,
                  w00 * fzi, w10 * fzi, w01 * fzi, w11 * fzi)
            acc = None
            for c8 in range(8):
                li = plsc.load_gather(low_bufs[c8], [ridx])
                col = lax.shift_left(li, 1) + par
                v = plsc.load_gather(row_bufs[c8], [ridx, col])
                acc = ws[c8] * v if acc is None else acc + ws[c8] * v
            plsc.store_scatter(outv, [ridx, colv], acc)
            return c

        lax.fori_loop(0, _CH // 8, p2, 0, unroll=False)

    for chunk in range(_NCH):
        base = wid * _PPW + chunk * _CH
        pltpu.sync_copy(xs_h.at[pl.ds(base, _CH)], xv)
        pltpu.sync_copy(ys_h.at[pl.ds(base, _CH)], yv)
        pltpu.sync_copy(zs_h.at[pl.ds(base, _CH)], zv)

        for lvl in range(3):
            res = _BASE_RES * (2 ** lvl)
            scale_f = jnp.float32(res - 1)
            size = _OFFSETS[lvl + 1] - _OFFSETS[lvl]
            process_level(lvl, scale_f, jnp.int32(_OFFSETS[lvl]),
                          jnp.int32(size - 1), False, jnp.int32(res))

        def lev_body(lvl, c):
            two = lax.shift_left(jnp.int32(1), lvl)
            scale_f = (two * _BASE_RES - 1).astype(jnp.float32)
            off = jnp.int32(_OFFSETS[3]) + (lvl - 3) * jnp.int32(_MAX_PARAMS)
            process_level(lvl, scale_f, off, jnp.int32(_MAX_PARAMS - 1),
                          True, None)
            return c

        lax.fori_loop(3, _NUM_LEVEL, lev_body, 0, unroll=False)

        pltpu.sync_copy(outv, out_h.at[pl.ds(base, _CH)])


_mesh = plsc.VectorSubcoreMesh(core_axis_name="c", subcore_axis_name="s",
                               num_cores=_NC, num_subcores=_NS)

_scratch = (
    [pltpu.VMEM((_CH,), jnp.float32) for _ in range(6)]
    + [pltpu.VMEM((_CH,), jnp.int32) for _ in range(8)]
    + [pltpu.VMEM((_CH,), jnp.int32) for _ in range(8)]
    + [pltpu.VMEM((_CH, 8), jnp.float32) for _ in range(8)]
    + [pltpu.VMEM((_CH, 2 * _NUM_LEVEL), jnp.float32)]
    + [pltpu.SemaphoreType.DMA]
)

_grid_kernel = pl.kernel(
    _body,
    out_type=jax.ShapeDtypeStruct((_N, 2 * _NUM_LEVEL), jnp.float32),
    mesh=_mesh,
    scratch_types=_scratch,
    compiler_params=pltpu.CompilerParams(needs_layout_passes=False,
                                         use_tc_tiling_on_sc=False),
)


def kernel(input_means, embeddings):
    ori_shape = input_means.shape[:-1]
    pts = input_means.reshape(-1, 3).T
    c0 = embeddings[:, 0]
    c1 = embeddings[:, 1]
    lo = lax.convert_element_type(lax.bitcast_convert_type(c0, jnp.uint32),
                                  jnp.uint64)
    hi = lax.convert_element_type(lax.bitcast_convert_type(c1, jnp.uint32),
                                  jnp.uint64)
    e64 = lax.bitwise_or(lo, lax.shift_left(hi, np.uint64(32)))
    emb8 = lax.bitcast_convert_type(e64.reshape(-1, 4), jnp.float32)
    emb8 = emb8.reshape(-1, 8)
    out = _grid_kernel(pts[0], pts[1], pts[2], emb8)
    return out.reshape(ori_shape + (2 * _NUM_LEVEL,))
